# Initial kernel scaffold; baseline (speedup 1.0000x reference)
import jax
import jax.numpy as jnp
from jax import lax
from jax.experimental import pallas as pl
from jax.experimental.pallas import tpu as pltpu
from jax.experimental.pallas import tpu_sc as plsc

_N = 10000
_NPAD = 10240
_G = 64
_NC = 2
_NS = 16
_NW = _NC * _NS
_CHUNK = 128
_RPS = _NPAD // _NS
_R = 512
_HP = lax.Precision.HIGHEST


def _sc_agg(with_deg, cpw):
  mesh = plsc.VectorSubcoreMesh(core_axis_name="c", subcore_axis_name="s")

  outs = [jax.ShapeDtypeStruct((_NC, _NPAD, 128), jnp.float32)]
  scr = [
      pltpu.VMEM((cpw, _CHUNK), jnp.int32),
      pltpu.VMEM((cpw, _CHUNK), jnp.int32),
      pltpu.VMEM((_CHUNK, 128), jnp.float32),
      pltpu.VMEM_SHARED((_NPAD, 128), jnp.float32),
  ]
  if with_deg:
    outs.append(jax.ShapeDtypeStruct((_NC, _NPAD, 16), jnp.float32))
    scr.append(pltpu.VMEM((_CHUNK, 16), jnp.float32))
    scr.append(pltpu.VMEM_SHARED((_NPAD, 16), jnp.float32))

  def body(*refs):
    if with_deg:
      (table, srcb, dstb, zrow, zdeg, ones_h,
       aggp, degp, idx_s, idx_d, rows, agg_sh, ones_v, deg_sh) = refs
    else:
      (table, srcb, dstb, zrow,
       aggp, idx_s, idx_d, rows, agg_sh) = refs

    cid = lax.axis_index("c")
    sid = lax.axis_index("s")
    wid = cid * _NS + sid

    @pl.loop(0, _RPS // _CHUNK)
    def _(z):
      pltpu.sync_copy(zrow, agg_sh.at[pl.ds(sid * _RPS + z * _CHUNK, _CHUNK)])
    if with_deg:
      @pl.loop(0, _RPS // _CHUNK)
      def _(z):
        pltpu.sync_copy(zdeg, deg_sh.at[pl.ds(sid * _RPS + z * _CHUNK, _CHUNK)])
      pltpu.sync_copy(ones_h, ones_v)
    pltpu.sync_copy(srcb.at[wid], idx_s)
    pltpu.sync_copy(dstb.at[wid], idx_d)
    plsc.subcore_barrier()

    @pl.loop(0, cpw)
    def _(j):
      pltpu.sync_copy(table.at[idx_s.at[j]], rows)
      pltpu.sync_copy(rows, agg_sh.at[idx_d.at[j]], add=True)
      if with_deg:
        pltpu.sync_copy(ones_v, deg_sh.at[idx_d.at[j]], add=True)

    plsc.subcore_barrier()

    sl = pl.ds(sid * _RPS, _RPS)
    pltpu.sync_copy(agg_sh.at[sl], aggp.at[cid].at[sl])
    if with_deg:
      pltpu.sync_copy(deg_sh.at[sl], degp.at[cid].at[sl])

  return pl.kernel(body, out_type=tuple(outs), mesh=mesh,
                   scratch_types=tuple(scr))


def _linear(x, w_t, b_row):
  m = x.shape[0]

  def body(x_ref, w_ref, b_ref, o_ref):
    o_ref[...] = (jnp.dot(x_ref[...], w_ref[...], precision=_HP)
                  + b_ref[...])

  return pl.pallas_call(
      body,
      grid=(m // _R,),
      in_specs=[
          pl.BlockSpec((_R, 128), lambda i: (i, 0)),
          pl.BlockSpec((128, 128), lambda i: (0, 0)),
          pl.BlockSpec((1, 128), lambda i: (0, 0)),
      ],
      out_specs=pl.BlockSpec((_R, 128), lambda i: (i, 0)),
      out_shape=jax.ShapeDtypeStruct((m, 128), jnp.float32),
  )(x, w_t, b_row)


def _deg_inv16(d_blk):
  d = d_blk[0] + d_blk[1]
  dinv = 1.0 / jnp.maximum(d, 1.0)
  return jnp.concatenate([dinv] * 8, axis=1)


def _sage_relu(p, degp, xr, wl_t):
  def body(p_ref, d_ref, xr_ref, w_ref, o_ref):
    agg = p_ref[0] + p_ref[1]
    t = jnp.dot(agg, w_ref[...], precision=_HP) * _deg_inv16(d_ref[...])
    o_ref[...] = jnp.maximum(t + xr_ref[...], 0.0)

  return pl.pallas_call(
      body,
      grid=(_NPAD // _R,),
      in_specs=[
          pl.BlockSpec((_NC, _R, 128), lambda i: (0, i, 0)),
          pl.BlockSpec((_NC, _R, 16), lambda i: (0, i, 0)),
          pl.BlockSpec((_R, 128), lambda i: (i, 0)),
          pl.BlockSpec((128, 128), lambda i: (0, 0)),
      ],
      out_specs=pl.BlockSpec((_R, 128), lambda i: (i, 0)),
      out_shape=jax.ShapeDtypeStruct((_NPAD, 128), jnp.float32),
  )(p, degp, xr, wl_t)


def _sage_pool(p, degp, hr, wl_t, batch_col):
  steps = _NPAD // _R

  def body(p_ref, d_ref, hr_ref, w_ref, b_ref, o_ref, acc, cnt):
    i = pl.program_id(0)

    @pl.when(i == 0)
    def _():
      acc[...] = jnp.zeros((_G, 128), jnp.float32)
      cnt[...] = jnp.zeros((_G, 128), jnp.float32)

    agg = p_ref[0] + p_ref[1]
    h2 = (jnp.dot(agg, w_ref[...], precision=_HP) * _deg_inv16(d_ref[...])
          + hr_ref[...])
    oh = (b_ref[...] == lax.broadcasted_iota(jnp.int32, (_R, _G), 1)
          ).astype(jnp.float32)
    acc[...] += lax.dot_general(oh, h2, (((0,), (0,)), ((), ())),
                                precision=_HP)
    cnt[...] += lax.dot_general(oh, jnp.ones((_R, 128), jnp.float32),
                                (((0,), (0,)), ((), ())), precision=_HP)

    @pl.when(i == steps - 1)
    def _():
      o_ref[...] = acc[...] / jnp.maximum(cnt[...], 1.0)

  return pl.pallas_call(
      body,
      grid=(steps,),
      in_specs=[
          pl.BlockSpec((_NC, _R, 128), lambda i: (0, i, 0)),
          pl.BlockSpec((_NC, _R, 16), lambda i: (0, i, 0)),
          pl.BlockSpec((_R, 128), lambda i: (i, 0)),
          pl.BlockSpec((128, 128), lambda i: (0, 0)),
          pl.BlockSpec((_R, 1), lambda i: (i, 0)),
      ],
      out_specs=pl.BlockSpec((_G, 128), lambda i: (0, 0)),
      out_shape=jax.ShapeDtypeStruct((_G, 128), jnp.float32),
      scratch_shapes=[
          pltpu.VMEM((_G, 128), jnp.float32),
          pltpu.VMEM((_G, 128), jnp.float32),
      ],
  )(p, degp, hr, wl_t, batch_col)


def kernel(x, edge_index, batch, W1_l, W1_r, b1, W2_l, W2_r, b2):
  e = edge_index.shape[1]
  cpw = -(-e // (_NW * _CHUNK))
  epad = _NW * cpw * _CHUNK

  src = edge_index[0]
  dst = edge_index[1]
  srcb = jnp.concatenate(
      [src, jnp.zeros((epad - e,), jnp.int32)]).reshape(_NW, cpw, _CHUNK)
  dstb = jnp.concatenate(
      [dst, jnp.full((epad - e,), _N, jnp.int32)]).reshape(_NW, cpw, _CHUNK)

  zrow = jnp.zeros((_CHUNK, 128), jnp.float32)
  zdeg = jnp.zeros((_CHUNK, 16), jnp.float32)
  ones_h = jnp.ones((_CHUNK, 16), jnp.float32)

  x_pad = jnp.pad(x, ((0, _NPAD - _N), (0, 0)))
  batch_col = jnp.concatenate(
      [batch, jnp.full((_NPAD - _N,), _G, jnp.int32)]).reshape(_NPAD, 1)

  sc_l1 = _sc_agg(True, cpw)
  sc_l2 = _sc_agg(False, cpw)

  xr = _linear(x_pad, W1_r.T, b1.reshape(1, 128))
  aggp1, degp = sc_l1(x, srcb, dstb, zrow, zdeg, ones_h)
  h1 = _sage_relu(aggp1, degp, xr, W1_l.T)

  hr = _linear(h1, W2_r.T, b2.reshape(1, 128))
  (aggp2,) = sc_l2(h1, srcb, dstb, zrow)

  return _sage_pool(aggp2, degp, hr, W2_l.T, batch_col)


# speedup vs baseline: 3.3935x; 3.3935x over previous
import dataclasses

import jax
import jax.numpy as jnp
from jax import lax
from jax.experimental import pallas as pl
from jax.experimental.pallas import tpu as pltpu
from jax.experimental.pallas import tpu_sc as plsc

_N = 10000
_NPAD = 10240
_G = 64
_NC = 2
_NS = 16
_NW = _NC * _NS
_CHUNK = 128
_IB = 8
_RPS = _NPAD // _NS
_DR = _NPAD // 128
_R = 512
_HP = lax.Precision.HIGHEST


def _sc_agg(with_deg, cpw):
  mesh = plsc.VectorSubcoreMesh(core_axis_name="c", subcore_axis_name="s")

  outs = [jax.ShapeDtypeStruct((_NC * _NPAD, 128), jnp.float32)]
  scr = [
      pltpu.VMEM((_IB, _CHUNK), jnp.int32),
      pltpu.VMEM((_IB, _CHUNK), jnp.int32),
      pltpu.VMEM((_CHUNK, 128), jnp.float32),
      pltpu.VMEM_SHARED((_NPAD + 128, 128), jnp.float32),
  ]
  if with_deg:
    outs.append(jax.ShapeDtypeStruct((_NC * _DR, 128), jnp.float32))
    scr.append(pltpu.VMEM((128, 128), jnp.float32))
    scr.append(pltpu.VMEM((_DR,), jnp.int32))

  def body(*refs):
    if with_deg:
      (table, srcb, dstb, zrow, i80,
       aggp, degp, idx_s, idx_d, rows, agg_sh, deg_t, i80_v) = refs
    else:
      (table, srcb, dstb, zrow,
       aggp, idx_s, idx_d, rows, agg_sh) = refs

    cid = lax.axis_index("c")
    sid = lax.axis_index("s")
    wid = cid * _NS + sid

    pltpu.sync_copy(zrow, rows)
    @pl.loop(0, _RPS // _CHUNK)
    def _(z):
      pltpu.sync_copy(rows, agg_sh.at[pl.ds(sid * _RPS + z * _CHUNK, _CHUNK)])
    if with_deg:
      pltpu.sync_copy(rows.at[pl.ds(0, 8)],
                      agg_sh.at[pl.ds(_NPAD + sid * 8, 8)])
      pltpu.sync_copy(zrow, deg_t)
      pltpu.sync_copy(i80, i80_v)
    plsc.subcore_barrier()

    ones16 = jnp.ones((16,), jnp.float32)

    @pl.loop(0, cpw // _IB)
    def _(jo):
      pltpu.sync_copy(srcb.at[pl.ds(wid * cpw + jo * _IB, _IB)], idx_s)
      pltpu.sync_copy(dstb.at[pl.ds(wid * cpw + jo * _IB, _IB)], idx_d)

      @pl.loop(0, _IB)
      def _(j):
        pltpu.sync_copy(table.at[idx_s.at[j]], rows)
        pltpu.sync_copy(rows, agg_sh.at[idx_d.at[j]], add=True)
        if with_deg:
          for k in range(_CHUNK // 16):
            dv = idx_d[j, pl.ds(k * 16, 16)]
            plsc.addupdate_scatter(
                deg_t, [lax.shift_right_logical(dv, 7), dv & 127], ones16)

    plsc.subcore_barrier()

    if with_deg:
      pltpu.sync_copy(deg_t.at[pl.ds(0, _DR)], agg_sh.at[i80_v], add=True)
      plsc.subcore_barrier()

    @pl.loop(0, _RPS // _CHUNK)
    def _(z):
      off = sid * _RPS + z * _CHUNK
      pltpu.sync_copy(agg_sh.at[pl.ds(off, _CHUNK)], rows)
      pltpu.sync_copy(rows, aggp.at[pl.ds(cid * _NPAD + off, _CHUNK)])
    if with_deg:
      @pl.when(sid < _DR // 8)
      def _():
        pltpu.sync_copy(agg_sh.at[pl.ds(_NPAD + sid * 8, 8)],
                        rows.at[pl.ds(0, 8)])
        pltpu.sync_copy(rows.at[pl.ds(0, 8)],
                        degp.at[pl.ds(cid * _DR + sid * 8, 8)])

  cp = pltpu.CompilerParams()
  if "needs_layout_passes" in pltpu.CompilerParams.__dataclass_fields__:
    cp = dataclasses.replace(cp, needs_layout_passes=False)
  return pl.kernel(body, out_type=tuple(outs), mesh=mesh,
                   scratch_types=tuple(scr), compiler_params=cp)


def _linear(x, w_t, b_row):
  m = x.shape[0]

  def body(x_ref, w_ref, b_ref, o_ref):
    o_ref[...] = (jnp.dot(x_ref[...], w_ref[...], precision=_HP)
                  + b_ref[...])

  return pl.pallas_call(
      body,
      grid=(m // _R,),
      in_specs=[
          pl.BlockSpec((_R, 128), lambda i: (i, 0)),
          pl.BlockSpec((128, 128), lambda i: (0, 0)),
          pl.BlockSpec((1, 128), lambda i: (0, 0)),
      ],
      out_specs=pl.BlockSpec((_R, 128), lambda i: (i, 0)),
      out_shape=jax.ShapeDtypeStruct((m, 128), jnp.float32),
  )(x, w_t, b_row)


def _sage_relu(p, dinv, xr, wl_t):
  def body(p_ref, d_ref, xr_ref, w_ref, o_ref):
    agg = (p_ref[0] + p_ref[1]) * d_ref[...]
    o_ref[...] = jnp.maximum(
        jnp.dot(agg, w_ref[...], precision=_HP) + xr_ref[...], 0.0)

  return pl.pallas_call(
      body,
      grid=(_NPAD // _R,),
      in_specs=[
          pl.BlockSpec((_NC, _R, 128), lambda i: (0, i, 0)),
          pl.BlockSpec((_R, 1), lambda i: (i, 0)),
          pl.BlockSpec((_R, 128), lambda i: (i, 0)),
          pl.BlockSpec((128, 128), lambda i: (0, 0)),
      ],
      out_specs=pl.BlockSpec((_R, 128), lambda i: (i, 0)),
      out_shape=jax.ShapeDtypeStruct((_NPAD, 128), jnp.float32),
  )(p, dinv, xr, wl_t)


def _sage_pool(p, dinv, hr, wl_t, batch_col):
  steps = _NPAD // _R

  def body(p_ref, d_ref, hr_ref, w_ref, b_ref, o_ref, acc, cnt):
    i = pl.program_id(0)

    @pl.when(i == 0)
    def _():
      acc[...] = jnp.zeros((_G, 128), jnp.float32)
      cnt[...] = jnp.zeros((_G, 128), jnp.float32)

    agg = (p_ref[0] + p_ref[1]) * d_ref[...]
    h2 = jnp.dot(agg, w_ref[...], precision=_HP) + hr_ref[...]
    oh = (b_ref[...] == lax.broadcasted_iota(jnp.int32, (_R, _G), 1)
          ).astype(jnp.float32)
    acc[...] += lax.dot_general(oh, h2, (((0,), (0,)), ((), ())),
                                precision=_HP)
    cnt[...] += lax.dot_general(oh, jnp.ones((_R, 128), jnp.float32),
                                (((0,), (0,)), ((), ())), precision=_HP)

    @pl.when(i == steps - 1)
    def _():
      o_ref[...] = acc[...] / jnp.maximum(cnt[...], 1.0)

  return pl.pallas_call(
      body,
      grid=(steps,),
      in_specs=[
          pl.BlockSpec((_NC, _R, 128), lambda i: (0, i, 0)),
          pl.BlockSpec((_R, 1), lambda i: (i, 0)),
          pl.BlockSpec((_R, 128), lambda i: (i, 0)),
          pl.BlockSpec((128, 128), lambda i: (0, 0)),
          pl.BlockSpec((_R, 1), lambda i: (i, 0)),
      ],
      out_specs=pl.BlockSpec((_G, 128), lambda i: (0, 0)),
      out_shape=jax.ShapeDtypeStruct((_G, 128), jnp.float32),
      scratch_shapes=[
          pltpu.VMEM((_G, 128), jnp.float32),
          pltpu.VMEM((_G, 128), jnp.float32),
      ],
  )(p, dinv, hr, wl_t, batch_col)


def kernel(x, edge_index, batch, W1_l, W1_r, b1, W2_l, W2_r, b2):
  e = edge_index.shape[1]
  cpw = -(-e // (_NW * _CHUNK * _IB)) * _IB
  epad = _NW * cpw * _CHUNK

  src = edge_index[0]
  dst = edge_index[1]
  srcb = jnp.concatenate(
      [src, jnp.zeros((epad - e,), jnp.int32)]).reshape(_NW * cpw, _CHUNK)
  dstb = jnp.concatenate(
      [dst, jnp.full((epad - e,), _N, jnp.int32)]).reshape(_NW * cpw, _CHUNK)

  zrow = jnp.zeros((_CHUNK, 128), jnp.float32)
  i80 = _NPAD + jnp.arange(_DR, dtype=jnp.int32)

  x_pad = jnp.pad(x, ((0, _NPAD - _N), (0, 0)))
  batch_col = jnp.concatenate(
      [batch, jnp.full((_NPAD - _N,), _G, jnp.int32)]).reshape(_NPAD, 1)

  sc_l1 = _sc_agg(True, cpw)
  sc_l2 = _sc_agg(False, cpw)

  xr = _linear(x_pad, W1_r.T, b1.reshape(1, 128))
  aggp1, degp = sc_l1(x, srcb, dstb, zrow, i80)
  aggp1 = aggp1.reshape(_NC, _NPAD, 128)

  deg = (degp[:_DR] + degp[_DR:]).reshape(_NPAD)
  dinv = (1.0 / jnp.maximum(deg, 1.0)).reshape(_NPAD, 1)

  h1 = _sage_relu(aggp1, dinv, xr, W1_l.T)

  hr = _linear(h1, W2_r.T, b2.reshape(1, 128))
  (aggp2,) = sc_l2(h1, srcb, dstb, zrow)
  aggp2 = aggp2.reshape(_NC, _NPAD, 128)

  return _sage_pool(aggp2, dinv, hr, W2_l.T, batch_col)
